# Initial kernel scaffold; baseline (speedup 1.0000x reference)
import jax
import jax.numpy as jnp
from jax.experimental import pallas as pl


def kernel(x, edge_index, edge_index_gt, edge_feats, emb_trigger, emb_action, W1l, b1, W1r, W2l, b2, W2r, Wp1, bp1, Wp2, bp2):
    raise NotImplementedError("write your pallas kernel here")


# speedup vs baseline: 6.7664x; 6.7664x over previous
import dataclasses
import functools

import jax
import jax.numpy as jnp
from jax import lax
from jax.experimental import pallas as pl
from jax.experimental.pallas import tpu as pltpu
from jax.experimental.pallas import tpu_sc as plsc

_N = 50000
_E = 800000
_G = 100000
_TRASH = 128
_NP = 50176
_EP = 819200
_ECH = 400
_ROWS_E = _EP // 128
_GP = 100352
_GCH = 196
_ROWS_G = _GP // 128
_ZTILE = _NP // 16
_WTILE = 3128

_mesh = plsc.VectorSubcoreMesh(core_axis_name="c", subcore_axis_name="s")
_f32 = jnp.float32
_i32 = jnp.int32

_cp_sc = pltpu.CompilerParams(use_tc_tiling_on_sc=False)
_cp_no_layout = dataclasses.replace(_cp_sc, needs_layout_passes=False)


def _zero_acc(acc, z_h, sub):
    pltpu.sync_copy(z_h, acc.at[pl.ds(sub * _ZTILE, 2048)])
    pltpu.sync_copy(z_h.at[pl.ds(0, _ZTILE - 2048)],
                    acc.at[pl.ds(sub * _ZTILE + 2048, _ZTILE - 2048)])


def _writeback(acc, out_h, sub):
    @pl.when(sub < 15)
    def _():
        pltpu.sync_copy(acc.at[pl.ds(sub * _WTILE, _WTILE)],
                        out_h.at[pl.ds(sub * _WTILE, _WTILE)])

    @pl.when(sub == 15)
    def _():
        pltpu.sync_copy(acc.at[pl.ds(15 * _WTILE, _N - 15 * _WTILE)],
                        out_h.at[pl.ds(15 * _WTILE, _N - 15 * _WTILE)])


def _sc_k1(src2, dst2, f02, f12, tabT, tabA, z16, ones8):
    @functools.partial(
        pl.kernel,
        out_type=[jax.ShapeDtypeStruct((_N, 16), _f32),
                  jax.ShapeDtypeStruct((_N, 16), _f32)],
        mesh=_mesh,
        scratch_types=[
            pltpu.VMEM((2048, 16), _f32),
            pltpu.VMEM((16, 128), _i32),
            pltpu.VMEM((16, 128), _i32),
            pltpu.VMEM((64, 4), _f32),
            pltpu.VMEM_SHARED((_NP, 16), _f32),
        ],
        compiler_params=_cp_no_layout,
    )
    def k1(src_h, dst_h, f0_h, f1_h, tabT_h, tabA_h, z_h, ones_h, outA, outB,
           rows_v, idx_v, fv, tab_v, acc):
        core = lax.axis_index("c")
        sub = lax.axis_index("s")

        def run(tab_h, node_h, feat_h, init_h, out_h, col0):
            pltpu.sync_copy(tab_h, tab_v)
            pltpu.sync_copy(init_h, rows_v)
            _zero_acc(acc, z_h, sub)
            plsc.subcore_barrier()

            @pl.loop(0, 25)
            def _(ci):
                c = ci * 16 + sub
                rbase = c * 16
                pltpu.sync_copy(node_h.at[pl.ds(rbase, 16)], idx_v)
                pltpu.sync_copy(feat_h.at[pl.ds(rbase, 16)], fv)

                @pl.loop(0, 16)
                def _(j):
                    for l in range(8):
                        fvec = fv[j, pl.ds(l * 16, 16)]
                        lanes = j * 128 + l * 16 + lax.iota(_i32, 16)
                        for jj in range(4):
                            vals = plsc.load_gather(
                                tab_v, [fvec, jnp.full((16,), jj, _i32)])
                            plsc.store_scatter(
                                rows_v,
                                [lanes, jnp.full((16,), col0 + jj, _i32)],
                                vals)

                @pl.loop(0, 16)
                def _(j):
                    pltpu.sync_copy(rows_v.at[pl.ds(j * 128, 128)],
                                    acc.at[idx_v.at[j]], add=True)

            plsc.subcore_barrier()
            _writeback(acc, out_h, sub)

        @pl.when(core == 0)
        def _():
            run(tabT_h, src_h, f0_h, z_h, outA, 0)

        @pl.when(core == 1)
        def _():
            run(tabA_h, dst_h, f1_h, ones_h, outB, 4)

    return k1(src2, dst2, f02, f12, tabT, tabA, z16, ones8)


def _sc_k3(x_pad, agg8, src2, dst2, z16):
    @functools.partial(
        pl.kernel,
        out_type=[jax.ShapeDtypeStruct((_N, 16), _f32),
                  jax.ShapeDtypeStruct((_N, 16), _f32)],
        mesh=_mesh,
        scratch_types=[
            pltpu.VMEM((2048, 16), _f32),
            pltpu.VMEM((16, 128), _i32),
            pltpu.VMEM((16, 128), _i32),
            pltpu.VMEM_SHARED((_NP, 16), _f32),
        ],
        compiler_params=_cp_sc,
    )
    def k3(x_h, a_h, src_h, dst_h, z_h, outx, outa,
           rows_v, idxs_v, idxd_v, acc):
        core = lax.axis_index("c")
        sub = lax.axis_index("s")

        def run(tab_h, out_h):
            _zero_acc(acc, z_h, sub)
            plsc.subcore_barrier()

            @pl.loop(0, 25)
            def _(ci):
                c = ci * 16 + sub
                rbase = c * 16
                pltpu.sync_copy(src_h.at[pl.ds(rbase, 16)], idxs_v)
                pltpu.sync_copy(dst_h.at[pl.ds(rbase, 16)], idxd_v)

                @pl.loop(0, 16)
                def _(j):
                    pltpu.sync_copy(tab_h.at[idxs_v.at[j]],
                                    rows_v.at[pl.ds(j * 128, 128)])

                @pl.loop(0, 16)
                def _(j):
                    pltpu.sync_copy(rows_v.at[pl.ds(j * 128, 128)],
                                    acc.at[idxd_v.at[j]], add=True)

            plsc.subcore_barrier()
            _writeback(acc, out_h, sub)

        @pl.when(core == 0)
        def _():
            run(x_h, outx)

        @pl.when(core == 1)
        def _():
            run(a_h, outa)

    return k3(x_pad, agg8, src2, dst2, z16)


def _sc_k5(g1s, src2, dst2, z32):
    @functools.partial(
        pl.kernel,
        out_type=[jax.ShapeDtypeStruct((_N, 32), _f32) for _ in range(4)],
        mesh=_mesh,
        scratch_types=[
            pltpu.VMEM((512, 32), _f32),
            pltpu.VMEM((4, 128), _i32),
            pltpu.VMEM((4, 128), _i32),
            pltpu.VMEM_SHARED((_NP, 32), _f32),
        ],
        compiler_params=_cp_sc,
    )
    def k5(g0_h, g1_h, g2_h, g3_h, src_h, dst_h, z_h, o0, o1, o2, o3,
           rows_v, idxs_v, idxd_v, acc):
        core = lax.axis_index("c")
        sub = lax.axis_index("s")

        def run(tab_h, out_h):
            _zero_acc(acc, z_h, sub)
            plsc.subcore_barrier()

            @pl.loop(0, 100)
            def _(ci):
                c = ci * 16 + sub
                rbase = c * 4
                pltpu.sync_copy(src_h.at[pl.ds(rbase, 4)], idxs_v)
                pltpu.sync_copy(dst_h.at[pl.ds(rbase, 4)], idxd_v)

                @pl.loop(0, 4)
                def _(j):
                    pltpu.sync_copy(tab_h.at[idxs_v.at[j]],
                                    rows_v.at[pl.ds(j * 128, 128)])

                @pl.loop(0, 4)
                def _(j):
                    pltpu.sync_copy(rows_v.at[pl.ds(j * 128, 128)],
                                    acc.at[idxd_v.at[j]], add=True)

            plsc.subcore_barrier()
            _writeback(acc, out_h, sub)

        @pl.when(core == 0)
        def _():
            run(g0_h, o0)
            run(g1_h, o1)

        @pl.when(core == 1)
        def _():
            run(g2_h, o2)
            run(g3_h, o3)

    return k5(g1s[0], g1s[1], g1s[2], g1s[3], src2, dst2, z32)


def _sc_k7(u, v, gts2, gtd2):
    @functools.partial(
        pl.kernel,
        out_type=[jax.ShapeDtypeStruct((_GP, 128), _f32),
                  jax.ShapeDtypeStruct((_GP, 128), _f32)],
        mesh=_mesh,
        scratch_types=[
            pltpu.VMEM((512, 128), _f32),
            pltpu.VMEM((4, 128), _i32),
        ],
        compiler_params=_cp_sc,
    )
    def k7(u_h, v_h, gts_h, gtd_h, hu_o, hv_o, buf_v, idx_v):
        core = lax.axis_index("c")
        sub = lax.axis_index("s")
        wid = core * 16 + sub

        for k in range(7):
            c = k * 32 + wid

            @pl.when(c < _GCH)
            def _():
                rbase = c * 4
                for tab_h, gt_h, out_h in ((u_h, gts_h, hu_o),
                                           (v_h, gtd_h, hv_o)):
                    pltpu.sync_copy(gt_h.at[pl.ds(rbase, 4)], idx_v)

                    @pl.loop(0, 4)
                    def _(j):
                        pltpu.sync_copy(tab_h.at[idx_v.at[j]],
                                        buf_v.at[pl.ds(j * 128, 128)])

                    pltpu.sync_copy(buf_v, out_h.at[pl.ds(c * 512, 512)])

    return k7(u, v, gts2, gtd2)


def _tc_combine(a, b):
    def body(a_r, b_r, o_r):
        o_r[...] = a_r[...] + b_r[...]

    return pl.pallas_call(
        body,
        grid=(25,),
        in_specs=[pl.BlockSpec((2000, 16), lambda i: (i, 0))] * 2,
        out_specs=pl.BlockSpec((2000, 16), lambda i: (i, 0)),
        out_shape=jax.ShapeDtypeStruct((_N, 16), _f32),
    )(a, b)


def _tc_layer1(s1x, s1a, x, agg8, W1lx, W1la, W1rx, W1ra, b1, W2lr, W2r, b2):
    def body(s1x_r, s1a_r, x_r, a8_r, wlx, wla, wrx, wra, b1r, w2l, w2r, b2r,
             g0, g1, g2, g3, r1o):
        rinv = 1.0 / jnp.maximum(a8_r[:, 8:9], 1.0)
        h1 = ((s1x_r[...] * rinv) @ wlx[...] + (s1a_r[...] * rinv) @ wla[...]
              + x_r[...] @ wrx[...] + a8_r[...] @ wra[...] + b1r[...])
        h1 = jnp.maximum(h1, 0.0)
        w2l_a = w2l[...]
        for k, o in enumerate((g0, g1, g2, g3)):
            o[...] = h1 @ w2l_a[k]
        r1o[...] = h1 @ w2r[...] + b2r[...]

    R = 2000
    row = lambda i: (i, 0)
    cst2 = lambda i: (0, 0)
    cst3 = lambda i: (0, 0, 0)
    return pl.pallas_call(
        body,
        grid=(_N // R,),
        in_specs=[
            pl.BlockSpec((R, 16), row), pl.BlockSpec((R, 16), row),
            pl.BlockSpec((R, 16), row), pl.BlockSpec((R, 16), row),
            pl.BlockSpec((16, 256), cst2), pl.BlockSpec((16, 256), cst2),
            pl.BlockSpec((16, 256), cst2), pl.BlockSpec((16, 256), cst2),
            pl.BlockSpec((1, 256), cst2),
            pl.BlockSpec((4, 256, 32), cst3),
            pl.BlockSpec((256, 128), cst2),
            pl.BlockSpec((1, 128), cst2),
        ],
        out_specs=[pl.BlockSpec((R, 32), row) for _ in range(4)]
        + [pl.BlockSpec((R, 128), row)],
        out_shape=[jax.ShapeDtypeStruct((_N, 32), _f32) for _ in range(4)]
        + [jax.ShapeDtypeStruct((_N, 128), _f32)],
    )(s1x, s1a, x, agg8, W1lx, W1la, W1rx, W1ra, b1, W2lr, W2r, b2)


def _tc_layer2(s2s, r1, agg8, Wp1t, Wp1b, bp1):
    def body(s0, s1, s2, s3, r1_r, a8_r, wt, wb, bp1r, u_o, v_o):
        rinv = 1.0 / jnp.maximum(a8_r[:, 8:9], 1.0)
        m = jnp.concatenate([s0[...], s1[...], s2[...], s3[...]], axis=1)
        h2 = jnp.maximum(m * rinv + r1_r[...], 0.0)
        u_o[...] = h2 @ wt[...] + bp1r[...]
        v_o[...] = h2 @ wb[...]

    R = 2000
    row = lambda i: (i, 0)
    cst2 = lambda i: (0, 0)
    return pl.pallas_call(
        body,
        grid=(_N // R,),
        in_specs=[pl.BlockSpec((R, 32), row) for _ in range(4)]
        + [pl.BlockSpec((R, 128), row), pl.BlockSpec((R, 16), row),
           pl.BlockSpec((128, 128), cst2), pl.BlockSpec((128, 128), cst2),
           pl.BlockSpec((1, 128), cst2)],
        out_specs=[pl.BlockSpec((R, 128), row)] * 2,
        out_shape=[jax.ShapeDtypeStruct((_N, 128), _f32)] * 2,
    )(s2s[0], s2s[1], s2s[2], s2s[3], r1, agg8, Wp1t, Wp1b, bp1)


def _tc_predict(hu, hv, Wp2, bp2):
    def body(hu_r, hv_r, w_r, b_r, o_r):
        z1 = jnp.maximum(hu_r[...] + hv_r[...], 0.0)
        o_r[...] = jax.nn.sigmoid(z1 @ w_r[...] + b_r[...])

    R = 2000
    row = lambda i: (i, 0)
    cst2 = lambda i: (0, 0)
    return pl.pallas_call(
        body,
        grid=(_G // R,),
        in_specs=[pl.BlockSpec((R, 128), row), pl.BlockSpec((R, 128), row),
                  pl.BlockSpec((128, 552), cst2), pl.BlockSpec((1, 552), cst2)],
        out_specs=pl.BlockSpec((R, 552), row),
        out_shape=jax.ShapeDtypeStruct((_G, 552), _f32),
    )(hu, hv, Wp2, bp2)


def kernel(x, edge_index, edge_index_gt, edge_feats, emb_trigger, emb_action,
           W1l, b1, W1r, W2l, b2, W2r, Wp1, bp1, Wp2, bp2):
    f32, i32 = _f32, _i32
    epad = _EP - _E
    trash = (_N + (jnp.arange(epad, dtype=i32) % _TRASH)).astype(i32)
    zpad = (jnp.arange(epad, dtype=i32) % 64).astype(i32)

    src = edge_index[0].astype(i32)
    dst = edge_index[1].astype(i32)
    src2 = jnp.concatenate([src, zpad]).reshape(_ROWS_E, 128)
    srct2 = jnp.concatenate([src, trash]).reshape(_ROWS_E, 128)
    dst2 = jnp.concatenate([dst, zpad]).reshape(_ROWS_E, 128)
    dstt2 = jnp.concatenate([dst, trash]).reshape(_ROWS_E, 128)
    f02 = jnp.concatenate([edge_feats[:, 0].astype(i32),
                           jnp.zeros((epad,), i32)]).reshape(_ROWS_E, 128)
    f12 = jnp.concatenate([edge_feats[:, 1].astype(i32),
                           jnp.zeros((epad,), i32)]).reshape(_ROWS_E, 128)

    gpad = _GP - _G
    gpad_idx = (jnp.arange(gpad, dtype=i32) % 64).astype(i32)
    gts2 = jnp.concatenate([edge_index_gt[0].astype(i32),
                            gpad_idx]).reshape(_ROWS_G, 128)
    gtd2 = jnp.concatenate([edge_index_gt[1].astype(i32),
                            gpad_idx]).reshape(_ROWS_G, 128)

    tabT = jnp.zeros((64, 4), f32).at[:45].set(emb_trigger.astype(f32))
    tabA = jnp.zeros((64, 4), f32).at[:47].set(emb_action.astype(f32))

    z16 = jnp.zeros((2048, 16), f32)
    z32 = jnp.zeros((2048, 32), f32)
    ones8 = jnp.zeros((2048, 16), f32).at[:, 8].set(1.0)

    accA, accB = _sc_k1(srct2, dstt2, f02, f12, tabT, tabA, z16, ones8)
    agg8 = _tc_combine(accA, accB)

    s1x, s1a = _sc_k3(x.astype(f32), agg8, src2, dstt2, z16)

    W1lx = W1l[:16].astype(f32)
    W1la = jnp.zeros((16, 256), f32).at[:8].set(W1l[16:24].astype(f32))
    W1rx = W1r[:16].astype(f32)
    W1ra = jnp.zeros((16, 256), f32).at[:8].set(W1r[16:24].astype(f32))
    W2lr = W2l.astype(f32).reshape(256, 4, 32).transpose(1, 0, 2)
    g1s = _tc_layer1(s1x, s1a, x.astype(f32), agg8, W1lx, W1la, W1rx, W1ra,
                     b1.astype(f32).reshape(1, 256), W2lr, W2r.astype(f32),
                     b2.astype(f32).reshape(1, 128))
    g1, r1 = g1s[:4], g1s[4]

    s2s = _sc_k5(g1, src2, dstt2, z32)

    u, v = _tc_layer2(s2s, r1, agg8, Wp1[:128].astype(f32),
                      Wp1[128:].astype(f32), bp1.astype(f32).reshape(1, 128))

    hu, hv = _sc_k7(u, v, gts2, gtd2)
    return _tc_predict(hu, hv, Wp2.astype(f32),
                       bp2.astype(f32).reshape(1, 552))


# speedup vs baseline: 9.3188x; 1.3772x over previous
import dataclasses
import functools

import jax
import jax.numpy as jnp
from jax import lax
from jax.experimental import pallas as pl
from jax.experimental.pallas import tpu as pltpu
from jax.experimental.pallas import tpu_sc as plsc

_N = 50000
_E = 800000
_G = 100000
_TRASH = 128
_NP = 50176
_EP = 819200
_ECH = 400
_ROWS_E = _EP // 128
_GP = 100352
_GCH = 196
_ROWS_G = _GP // 128
_ZTILE = _NP // 16
_WTILE = 3128

_mesh = plsc.VectorSubcoreMesh(core_axis_name="c", subcore_axis_name="s")
_f32 = jnp.float32
_i32 = jnp.int32

_cp_sc = pltpu.CompilerParams(use_tc_tiling_on_sc=False)
_cp_no_layout = dataclasses.replace(_cp_sc, needs_layout_passes=False)


def _zero_acc(acc, z_h, sub):
    pltpu.sync_copy(z_h, acc.at[pl.ds(sub * _ZTILE, 2048)])
    pltpu.sync_copy(z_h.at[pl.ds(0, _ZTILE - 2048)],
                    acc.at[pl.ds(sub * _ZTILE + 2048, _ZTILE - 2048)])


def _seg_pipeline(tab_h, src_h, dst_h, acc, rows0, rows1, idxs_v, idxd_v,
                  sg0, sg1, ss0, ss1, sub):
    rowbufs = (rows0, rows1)
    sgs = (sg0, sg1)
    sss = (ss0, ss1)
    g_h = [None, None]
    s_h = [None, None]

    def gfire(b, j):
        g_h[b] = [
            pltpu.async_copy(tab_h.at[idxs_v.at[2 * j + r]],
                             rowbufs[b].at[pl.ds(r * 128, 128)], sgs[b])
            for r in range(2)
        ]

    def gwait(b):
        for h in g_h[b]:
            h.wait()

    def sfire(b, j):
        s_h[b] = [
            pltpu.async_copy(rowbufs[b].at[pl.ds(r * 128, 128)],
                             acc.at[idxd_v.at[2 * j + r]], sss[b], add=True)
            for r in range(2)
        ]

    def swait(b):
        for h in s_h[b]:
            h.wait()

    @pl.loop(0, 50)
    def _(si):
        s = si * 16 + sub
        rbase = s * 8
        pltpu.sync_copy(src_h.at[pl.ds(rbase, 8)], idxs_v)
        pltpu.sync_copy(dst_h.at[pl.ds(rbase, 8)], idxd_v)
        gfire(0, 0)
        for j in range(4):
            b = j % 2
            if j > 0:
                swait(1 - b)
            if j + 1 < 4:
                gfire(1 - b, j + 1)
            gwait(b)
            sfire(b, j)
        swait(1)


def _writeback(acc, out_h, sub):
    @pl.when(sub < 15)
    def _():
        pltpu.sync_copy(acc.at[pl.ds(sub * _WTILE, _WTILE)],
                        out_h.at[pl.ds(sub * _WTILE, _WTILE)])

    @pl.when(sub == 15)
    def _():
        pltpu.sync_copy(acc.at[pl.ds(15 * _WTILE, _N - 15 * _WTILE)],
                        out_h.at[pl.ds(15 * _WTILE, _N - 15 * _WTILE)])


def _sc_k1(src2, dst2, f02, f12, tabT, tabA, z16, ones8):
    @functools.partial(
        pl.kernel,
        out_type=[jax.ShapeDtypeStruct((_N, 16), _f32),
                  jax.ShapeDtypeStruct((_N, 16), _f32)],
        mesh=_mesh,
        scratch_types=[
            pltpu.VMEM((2048, 16), _f32),
            pltpu.VMEM((16, 128), _i32),
            pltpu.VMEM((16, 128), _i32),
            pltpu.VMEM((64, 4), _f32),
            pltpu.VMEM_SHARED((_NP, 16), _f32),
        ],
        compiler_params=_cp_no_layout,
    )
    def k1(src_h, dst_h, f0_h, f1_h, tabT_h, tabA_h, z_h, ones_h, outA, outB,
           rows_v, idx_v, fv, tab_v, acc):
        core = lax.axis_index("c")
        sub = lax.axis_index("s")

        def run(tab_h, node_h, feat_h, init_h, out_h, col0):
            pltpu.sync_copy(tab_h, tab_v)
            pltpu.sync_copy(init_h, rows_v)
            _zero_acc(acc, z_h, sub)
            plsc.subcore_barrier()

            @pl.loop(0, 25)
            def _(ci):
                c = ci * 16 + sub
                rbase = c * 16
                pltpu.sync_copy(node_h.at[pl.ds(rbase, 16)], idx_v)
                pltpu.sync_copy(feat_h.at[pl.ds(rbase, 16)], fv)

                @pl.loop(0, 16)
                def _(j):
                    for l in range(8):
                        fvec = fv[j, pl.ds(l * 16, 16)]
                        lanes = j * 128 + l * 16 + lax.iota(_i32, 16)
                        for jj in range(4):
                            vals = plsc.load_gather(
                                tab_v, [fvec, jnp.full((16,), jj, _i32)])
                            plsc.store_scatter(
                                rows_v,
                                [lanes, jnp.full((16,), col0 + jj, _i32)],
                                vals)

                @pl.loop(0, 16)
                def _(j):
                    pltpu.sync_copy(rows_v.at[pl.ds(j * 128, 128)],
                                    acc.at[idx_v.at[j]], add=True)

            plsc.subcore_barrier()
            _writeback(acc, out_h, sub)

        @pl.when(core == 0)
        def _():
            run(tabT_h, src_h, f0_h, z_h, outA, 0)

        @pl.when(core == 1)
        def _():
            run(tabA_h, dst_h, f1_h, ones_h, outB, 4)

    return k1(src2, dst2, f02, f12, tabT, tabA, z16, ones8)


def _sc_k3(x_pad, agg8, src2, dst2, z16):
    @functools.partial(
        pl.kernel,
        out_type=[jax.ShapeDtypeStruct((_N, 16), _f32),
                  jax.ShapeDtypeStruct((_N, 16), _f32)],
        mesh=_mesh,
        scratch_types=[
            pltpu.VMEM((256, 16), _f32),
            pltpu.VMEM((256, 16), _f32),
            pltpu.VMEM((8, 128), _i32),
            pltpu.VMEM((8, 128), _i32),
            pltpu.VMEM_SHARED((_NP, 16), _f32),
            pltpu.SemaphoreType.DMA, pltpu.SemaphoreType.DMA,
            pltpu.SemaphoreType.DMA, pltpu.SemaphoreType.DMA,
        ],
        compiler_params=_cp_sc,
    )
    def k3(x_h, a_h, src_h, dst_h, z_h, outx, outa,
           rows0, rows1, idxs_v, idxd_v, acc, sg0, sg1, ss0, ss1):
        core = lax.axis_index("c")
        sub = lax.axis_index("s")

        def run(tab_h, out_h):
            _zero_acc(acc, z_h, sub)
            plsc.subcore_barrier()
            _seg_pipeline(tab_h, src_h, dst_h, acc, rows0, rows1,
                          idxs_v, idxd_v, sg0, sg1, ss0, ss1, sub)
            plsc.subcore_barrier()
            _writeback(acc, out_h, sub)

        @pl.when(core == 0)
        def _():
            run(x_h, outx)

        @pl.when(core == 1)
        def _():
            run(a_h, outa)

    return k3(x_pad, agg8, src2, dst2, z16)


def _sc_k5(g1s, src2, dst2, z32):
    @functools.partial(
        pl.kernel,
        out_type=[jax.ShapeDtypeStruct((_N, 32), _f32) for _ in range(4)],
        mesh=_mesh,
        scratch_types=[
            pltpu.VMEM((256, 32), _f32),
            pltpu.VMEM((256, 32), _f32),
            pltpu.VMEM((8, 128), _i32),
            pltpu.VMEM((8, 128), _i32),
            pltpu.VMEM_SHARED((_NP, 32), _f32),
            pltpu.SemaphoreType.DMA, pltpu.SemaphoreType.DMA,
            pltpu.SemaphoreType.DMA, pltpu.SemaphoreType.DMA,
        ],
        compiler_params=_cp_sc,
    )
    def k5(g0_h, g1_h, g2_h, g3_h, src_h, dst_h, z_h, o0, o1, o2, o3,
           rows0, rows1, idxs_v, idxd_v, acc, sg0, sg1, ss0, ss1):
        core = lax.axis_index("c")
        sub = lax.axis_index("s")

        def run(tab_h, out_h):
            _zero_acc(acc, z_h, sub)
            plsc.subcore_barrier()
            _seg_pipeline(tab_h, src_h, dst_h, acc, rows0, rows1,
                          idxs_v, idxd_v, sg0, sg1, ss0, ss1, sub)
            plsc.subcore_barrier()
            _writeback(acc, out_h, sub)

        @pl.when(core == 0)
        def _():
            run(g0_h, o0)
            run(g1_h, o1)

        @pl.when(core == 1)
        def _():
            run(g2_h, o2)
            run(g3_h, o3)

    return k5(g1s[0], g1s[1], g1s[2], g1s[3], src2, dst2, z32)


def _sc_k7(u, v, gts2, gtd2):
    @functools.partial(
        pl.kernel,
        out_type=[jax.ShapeDtypeStruct((_GP, 128), _f32),
                  jax.ShapeDtypeStruct((_GP, 128), _f32)],
        mesh=_mesh,
        scratch_types=[
            pltpu.VMEM((512, 128), _f32),
            pltpu.VMEM((4, 128), _i32),
        ],
        compiler_params=_cp_sc,
    )
    def k7(u_h, v_h, gts_h, gtd_h, hu_o, hv_o, buf_v, idx_v):
        core = lax.axis_index("c")
        sub = lax.axis_index("s")
        wid = core * 16 + sub

        for k in range(7):
            c = k * 32 + wid

            @pl.when(c < _GCH)
            def _():
                rbase = c * 4
                for tab_h, gt_h, out_h in ((u_h, gts_h, hu_o),
                                           (v_h, gtd_h, hv_o)):
                    pltpu.sync_copy(gt_h.at[pl.ds(rbase, 4)], idx_v)

                    @pl.loop(0, 4)
                    def _(j):
                        pltpu.sync_copy(tab_h.at[idx_v.at[j]],
                                        buf_v.at[pl.ds(j * 128, 128)])

                    pltpu.sync_copy(buf_v, out_h.at[pl.ds(c * 512, 512)])

    return k7(u, v, gts2, gtd2)


def _tc_combine(a, b):
    def body(a_r, b_r, o_r):
        o_r[...] = a_r[...] + b_r[...]

    return pl.pallas_call(
        body,
        grid=(25,),
        in_specs=[pl.BlockSpec((2000, 16), lambda i: (i, 0))] * 2,
        out_specs=pl.BlockSpec((2000, 16), lambda i: (i, 0)),
        out_shape=jax.ShapeDtypeStruct((_N, 16), _f32),
    )(a, b)


def _tc_layer1(s1x, s1a, x, agg8, W1lx, W1la, W1rx, W1ra, b1, W2lr, W2r, b2):
    def body(s1x_r, s1a_r, x_r, a8_r, wlx, wla, wrx, wra, b1r, w2l, w2r, b2r,
             g0, g1, g2, g3, r1o):
        rinv = 1.0 / jnp.maximum(a8_r[:, 8:9], 1.0)
        h1 = ((s1x_r[...] * rinv) @ wlx[...] + (s1a_r[...] * rinv) @ wla[...]
              + x_r[...] @ wrx[...] + a8_r[...] @ wra[...] + b1r[...])
        h1 = jnp.maximum(h1, 0.0)
        w2l_a = w2l[...]
        for k, o in enumerate((g0, g1, g2, g3)):
            o[...] = h1 @ w2l_a[k]
        r1o[...] = h1 @ w2r[...] + b2r[...]

    R = 2000
    row = lambda i: (i, 0)
    cst2 = lambda i: (0, 0)
    cst3 = lambda i: (0, 0, 0)
    return pl.pallas_call(
        body,
        grid=(_N // R,),
        in_specs=[
            pl.BlockSpec((R, 16), row), pl.BlockSpec((R, 16), row),
            pl.BlockSpec((R, 16), row), pl.BlockSpec((R, 16), row),
            pl.BlockSpec((16, 256), cst2), pl.BlockSpec((16, 256), cst2),
            pl.BlockSpec((16, 256), cst2), pl.BlockSpec((16, 256), cst2),
            pl.BlockSpec((1, 256), cst2),
            pl.BlockSpec((4, 256, 32), cst3),
            pl.BlockSpec((256, 128), cst2),
            pl.BlockSpec((1, 128), cst2),
        ],
        out_specs=[pl.BlockSpec((R, 32), row) for _ in range(4)]
        + [pl.BlockSpec((R, 128), row)],
        out_shape=[jax.ShapeDtypeStruct((_N, 32), _f32) for _ in range(4)]
        + [jax.ShapeDtypeStruct((_N, 128), _f32)],
    )(s1x, s1a, x, agg8, W1lx, W1la, W1rx, W1ra, b1, W2lr, W2r, b2)


def _tc_layer2(s2s, r1, agg8, Wp1t, Wp1b, bp1):
    def body(s0, s1, s2, s3, r1_r, a8_r, wt, wb, bp1r, u_o, v_o):
        rinv = 1.0 / jnp.maximum(a8_r[:, 8:9], 1.0)
        m = jnp.concatenate([s0[...], s1[...], s2[...], s3[...]], axis=1)
        h2 = jnp.maximum(m * rinv + r1_r[...], 0.0)
        u_o[...] = h2 @ wt[...] + bp1r[...]
        v_o[...] = h2 @ wb[...]

    R = 2000
    row = lambda i: (i, 0)
    cst2 = lambda i: (0, 0)
    return pl.pallas_call(
        body,
        grid=(_N // R,),
        in_specs=[pl.BlockSpec((R, 32), row) for _ in range(4)]
        + [pl.BlockSpec((R, 128), row), pl.BlockSpec((R, 16), row),
           pl.BlockSpec((128, 128), cst2), pl.BlockSpec((128, 128), cst2),
           pl.BlockSpec((1, 128), cst2)],
        out_specs=[pl.BlockSpec((R, 128), row)] * 2,
        out_shape=[jax.ShapeDtypeStruct((_N, 128), _f32)] * 2,
    )(s2s[0], s2s[1], s2s[2], s2s[3], r1, agg8, Wp1t, Wp1b, bp1)


def _tc_predict(hu, hv, Wp2, bp2):
    def body(hu_r, hv_r, w_r, b_r, o_r):
        z1 = jnp.maximum(hu_r[...] + hv_r[...], 0.0)
        o_r[...] = jax.nn.sigmoid(z1 @ w_r[...] + b_r[...])

    R = 2000
    row = lambda i: (i, 0)
    cst2 = lambda i: (0, 0)
    return pl.pallas_call(
        body,
        grid=(_G // R,),
        in_specs=[pl.BlockSpec((R, 128), row), pl.BlockSpec((R, 128), row),
                  pl.BlockSpec((128, 552), cst2), pl.BlockSpec((1, 552), cst2)],
        out_specs=pl.BlockSpec((R, 552), row),
        out_shape=jax.ShapeDtypeStruct((_G, 552), _f32),
    )(hu, hv, Wp2, bp2)


def kernel(x, edge_index, edge_index_gt, edge_feats, emb_trigger, emb_action,
           W1l, b1, W1r, W2l, b2, W2r, Wp1, bp1, Wp2, bp2):
    f32, i32 = _f32, _i32
    epad = _EP - _E
    trash = (_N + (jnp.arange(epad, dtype=i32) % _TRASH)).astype(i32)
    zpad = (jnp.arange(epad, dtype=i32) % 64).astype(i32)

    src = edge_index[0].astype(i32)
    dst = edge_index[1].astype(i32)
    src2 = jnp.concatenate([src, zpad]).reshape(_ROWS_E, 128)
    srct2 = jnp.concatenate([src, trash]).reshape(_ROWS_E, 128)
    dst2 = jnp.concatenate([dst, zpad]).reshape(_ROWS_E, 128)
    dstt2 = jnp.concatenate([dst, trash]).reshape(_ROWS_E, 128)
    f02 = jnp.concatenate([edge_feats[:, 0].astype(i32),
                           jnp.zeros((epad,), i32)]).reshape(_ROWS_E, 128)
    f12 = jnp.concatenate([edge_feats[:, 1].astype(i32),
                           jnp.zeros((epad,), i32)]).reshape(_ROWS_E, 128)

    gpad = _GP - _G
    gpad_idx = (jnp.arange(gpad, dtype=i32) % 64).astype(i32)
    gts2 = jnp.concatenate([edge_index_gt[0].astype(i32),
                            gpad_idx]).reshape(_ROWS_G, 128)
    gtd2 = jnp.concatenate([edge_index_gt[1].astype(i32),
                            gpad_idx]).reshape(_ROWS_G, 128)

    tabT = jnp.zeros((64, 4), f32).at[:45].set(emb_trigger.astype(f32))
    tabA = jnp.zeros((64, 4), f32).at[:47].set(emb_action.astype(f32))

    z16 = jnp.zeros((2048, 16), f32)
    z32 = jnp.zeros((2048, 32), f32)
    ones8 = jnp.zeros((2048, 16), f32).at[:, 8].set(1.0)

    accA, accB = _sc_k1(srct2, dstt2, f02, f12, tabT, tabA, z16, ones8)
    agg8 = _tc_combine(accA, accB)

    s1x, s1a = _sc_k3(x.astype(f32), agg8, src2, dstt2, z16)

    W1lx = W1l[:16].astype(f32)
    W1la = jnp.zeros((16, 256), f32).at[:8].set(W1l[16:24].astype(f32))
    W1rx = W1r[:16].astype(f32)
    W1ra = jnp.zeros((16, 256), f32).at[:8].set(W1r[16:24].astype(f32))
    W2lr = W2l.astype(f32).reshape(256, 4, 32).transpose(1, 0, 2)
    g1s = _tc_layer1(s1x, s1a, x.astype(f32), agg8, W1lx, W1la, W1rx, W1ra,
                     b1.astype(f32).reshape(1, 256), W2lr, W2r.astype(f32),
                     b2.astype(f32).reshape(1, 128))
    g1, r1 = g1s[:4], g1s[4]

    s2s = _sc_k5(g1, src2, dstt2, z32)

    u, v = _tc_layer2(s2s, r1, agg8, Wp1[:128].astype(f32),
                      Wp1[128:].astype(f32), bp1.astype(f32).reshape(1, 128))

    hu, hv = _sc_k7(u, v, gts2, gtd2)
    return _tc_predict(hu, hv, Wp2.astype(f32),
                       bp2.astype(f32).reshape(1, 552))
